# baseline (speedup 1.0000x reference)
import jax
import jax.numpy as jnp
from jax import lax
from jax.experimental import pallas as pl
from jax.experimental.pallas import tpu as pltpu
from jax.experimental.pallas import tpu_sc as plsc

BATCH = 16384
EMBED = 16
NUM = 1000000
_NC = 2
_NS = 16
_NW = _NC * _NS
_CHUNK = BATCH // _NW
_PASS = _CHUNK // 2

_SLAB = 1024
_SLABP = 1033
_NSLAB = 976
_MAIN = _NSLAB * _SLAB
_TAILW = 640
_TAIL0 = NUM - _TAILW


def _shuffle2(u_slab, i_slab, uo_v, io_v):
    row_iota = lax.iota(jnp.int32, 16)
    zeros = jnp.full((16,), 0, jnp.int32)

    def step(r, carry):
        c0 = r * 16
        cols = [zeros + (c0 + s) for s in range(16)]
        uvs = [plsc.load_gather(u_slab, [row_iota, cols[s]])
               for s in range(16)]
        ivs = [plsc.load_gather(i_slab, [row_iota, cols[s]])
               for s in range(16)]
        for s in range(16):
            uo_v[r * 2 + (s >> 3), pl.ds((s & 7) * 16, 16)] = uvs[s]
            io_v[r * 2 + (s >> 3), pl.ds((s & 7) * 16, 16)] = ivs[s]
        return carry

    lax.fori_loop(0, _SLAB // 16, step, 0)


def _fire(uft_hbm, ift_hbm, u_b, i_b, sem, sl):
    c0 = sl * _SLAB
    pltpu.async_copy(
        uft_hbm.at[:, pl.ds(c0, _SLAB)], u_b.at[:, pl.ds(0, _SLAB)], sem)
    pltpu.async_copy(
        ift_hbm.at[:, pl.ds(c0, _SLAB)], i_b.at[:, pl.ds(0, _SLAB)], sem)


def _drain(uft_hbm, ift_hbm, u_b, i_b, sem, sl):
    c0 = sl * _SLAB
    pltpu.make_async_copy(
        uft_hbm.at[:, pl.ds(c0, _SLAB)], u_b.at[:, pl.ds(0, _SLAB)],
        sem).wait()
    pltpu.make_async_copy(
        ift_hbm.at[:, pl.ds(c0, _SLAB)], i_b.at[:, pl.ds(0, _SLAB)],
        sem).wait()


def _detrans_body(uft_hbm, ift_hbm, usup_hbm, isup_hbm,
                  u_s0, i_s0, u_s1, i_s1, uo_v, io_v, sem0, sem1, semw):
    wid = lax.axis_index("s") * _NC + lax.axis_index("c")
    bufs = ((u_s0, i_s0, sem0), (u_s1, i_s1, sem1))
    nrow = _SLAB // 8

    def _drain_writes(sl_prev):
        r0p = sl_prev * nrow
        pltpu.make_async_copy(
            uo_v, usup_hbm.at[pl.ds(r0p, nrow)], semw).wait()
        pltpu.make_async_copy(
            io_v, isup_hbm.at[pl.ds(r0p, nrow)], semw).wait()

    _fire(uft_hbm, ift_hbm, u_s0, i_s0, sem0, wid)

    def pair(k, carry):
        for e in range(2):
            j = k * 2 + e
            sl = j * _NW + wid
            u_b, i_b, sem = bufs[e]
            nu_b, ni_b, nsem = bufs[e ^ 1]

            @pl.when(sl < _NSLAB)
            def _(sl=sl, u_b=u_b, i_b=i_b, sem=sem,
                  nu_b=nu_b, ni_b=ni_b, nsem=nsem):
                _drain(uft_hbm, ift_hbm, u_b, i_b, sem, sl)

                @pl.when(sl + _NW < _NSLAB)
                def _():
                    _fire(uft_hbm, ift_hbm, nu_b, ni_b, nsem, sl + _NW)

                @pl.when(sl >= _NW)
                def _():
                    _drain_writes(sl - _NW)

                r0 = sl * nrow
                _shuffle2(u_b, i_b, uo_v, io_v)
                pltpu.async_copy(uo_v, usup_hbm.at[pl.ds(r0, nrow)], semw)
                pltpu.async_copy(io_v, isup_hbm.at[pl.ds(r0, nrow)], semw)

        return carry

    lax.fori_loop(0, ((_NSLAB + _NW - 1) // _NW + 1) // 2, pair, 0)
    _drain_writes(((_NSLAB - 1 - wid) // _NW) * _NW + wid)


def _gather_body(uid_hbm, iid_hbm, uf_hbm, if_hbm, ut_hbm, it_hbm,
                 ub_hbm, ib_hbm, gb_hbm,
                 pred_hbm, cvr_hbm,
                 uid_v, iid_v, usup_v, isup_v, u_rows, i_rows, ut_v, it_v,
                 ub_v, ib_v, pred_v, cvr_v, gb_v, sem):
    wid = lax.axis_index("s") * _NC + lax.axis_index("c")
    base = wid * _CHUNK

    pltpu.sync_copy(uid_hbm.at[pl.ds(base, _CHUNK)], uid_v)
    pltpu.sync_copy(iid_hbm.at[pl.ds(base, _CHUNK)], iid_v)
    pltpu.sync_copy(gb_hbm, gb_v)
    pltpu.sync_copy(ut_hbm, ut_v)
    pltpu.sync_copy(it_hbm, it_v)

    def supidx(j, carry):
        s = j * 16
        usup_v[pl.ds(s, 16)] = lax.shift_right_logical(uid_v[pl.ds(s, 16)], 3)
        isup_v[pl.ds(s, 16)] = lax.shift_right_logical(iid_v[pl.ds(s, 16)], 3)
        return carry

    lax.fori_loop(0, _CHUNK // 16, supidx, 0)

    cp_ub = pltpu.async_copy(ub_hbm.at[uid_v], ub_v, sem)
    cp_ib = pltpu.async_copy(ib_hbm.at[iid_v], ib_v, sem)

    gb_vec = gb_v[...]

    for p in range(2):
        cp_u = pltpu.async_copy(
            uf_hbm.at[usup_v.at[pl.ds(p * _PASS, _PASS)]], u_rows, sem)
        cp_i = pltpu.async_copy(
            if_hbm.at[isup_v.at[pl.ds(p * _PASS, _PASS)]], i_rows, sem)
        cp_u.wait()
        cp_i.wait()
        if p == 0:
            cp_ub.wait()
            cp_ib.wait()

        def block(j, carry):
            b16 = p * _PASS + j * 16
            row_idx = lax.iota(jnp.int32, 16) + j * 16
            uid_b = uid_v[pl.ds(b16, 16)]
            iid_b = iid_v[pl.ds(b16, 16)]
            ucol = (uid_b & 7) * 16
            icol = (iid_b & 7) * 16
            ut_col = jnp.maximum(uid_b - _TAIL0, 0)
            it_col = jnp.maximum(iid_b - _TAIL0, 0)
            u_tail = uid_b >= _MAIN
            i_tail = iid_b >= _MAIN
            acc = ub_v[pl.ds(b16, 16)] + ib_v[pl.ds(b16, 16)] + gb_vec
            for d in range(EMBED):
                dvec = jnp.full((16,), d, jnp.int32)
                um = plsc.load_gather(u_rows, [row_idx, ucol + d])
                im = plsc.load_gather(i_rows, [row_idx, icol + d])
                ut = plsc.load_gather(ut_v, [dvec, ut_col])
                it = plsc.load_gather(it_v, [dvec, it_col])
                uu = jnp.where(u_tail, ut, um)
                ii = jnp.where(i_tail, it, im)
                acc = acc + uu * ii
            pred_v[pl.ds(b16, 16)] = acc
            cvr_v[pl.ds(b16, 16)] = 1.0 / (1.0 + jnp.exp(-acc))
            return carry

        lax.fori_loop(0, _PASS // 16, block, 0)

    pltpu.sync_copy(pred_v, pred_hbm.at[pl.ds(base, _CHUNK)])
    pltpu.sync_copy(cvr_v, cvr_hbm.at[pl.ds(base, _CHUNK)])


@jax.jit
def _run(user_id, item_id, user_factors, item_factors, user_bias, item_bias,
         gb16):
    f32 = jnp.float32
    mesh = plsc.VectorSubcoreMesh(core_axis_name="c", subcore_axis_name="s")
    params = pltpu.CompilerParams(
        needs_layout_passes=False, disable_bounds_checks=True)

    detrans = pl.kernel(
        _detrans_body,
        out_type=(jax.ShapeDtypeStruct((NUM // 8, 8 * EMBED), f32),
                  jax.ShapeDtypeStruct((NUM // 8, 8 * EMBED), f32)),
        mesh=mesh,
        compiler_params=params,
        scratch_types=[
            pltpu.VMEM((EMBED, _SLABP), f32),
            pltpu.VMEM((EMBED, _SLABP), f32),
            pltpu.VMEM((EMBED, _SLABP), f32),
            pltpu.VMEM((EMBED, _SLABP), f32),
            pltpu.VMEM((_SLAB // 8, 128), f32),
            pltpu.VMEM((_SLAB // 8, 128), f32),
            pltpu.SemaphoreType.DMA,
            pltpu.SemaphoreType.DMA,
            pltpu.SemaphoreType.DMA,
        ],
    )
    usup, isup = detrans(user_factors.T, item_factors.T)

    gather = pl.kernel(
        _gather_body,
        out_type=(jax.ShapeDtypeStruct((BATCH,), f32),
                  jax.ShapeDtypeStruct((BATCH,), f32)),
        mesh=mesh,
        compiler_params=params,
        scratch_types=[
            pltpu.VMEM((_CHUNK,), jnp.int32),
            pltpu.VMEM((_CHUNK,), jnp.int32),
            pltpu.VMEM((_CHUNK,), jnp.int32),
            pltpu.VMEM((_CHUNK,), jnp.int32),
            pltpu.VMEM((_PASS, 128), f32),
            pltpu.VMEM((_PASS, 128), f32),
            pltpu.VMEM((EMBED, _TAILW), f32),
            pltpu.VMEM((EMBED, _TAILW), f32),
            pltpu.VMEM((_CHUNK,), f32),
            pltpu.VMEM((_CHUNK,), f32),
            pltpu.VMEM((_CHUNK,), f32),
            pltpu.VMEM((_CHUNK,), f32),
            pltpu.VMEM((16,), f32),
            pltpu.SemaphoreType.DMA,
        ],
    )
    ut = user_factors[_TAIL0:].T
    it = item_factors[_TAIL0:].T
    return gather(user_id, item_id, usup, isup, ut, it,
                  user_bias, item_bias, gb16)


def kernel(user_id, item_id, user_factors, item_factors, user_bias,
           item_bias, global_bias):
    gb16 = jnp.broadcast_to(global_bias.astype(jnp.float32), (16,))
    pred, cvr = _run(user_id.astype(jnp.int32), item_id.astype(jnp.int32),
                     user_factors, item_factors, user_bias, item_bias, gb16)
    return (pred, cvr)


# speedup vs baseline: 1.1026x; 1.1026x over previous
import jax
import jax.numpy as jnp
from jax import lax
from jax.experimental import pallas as pl
from jax.experimental.pallas import tpu as pltpu
from jax.experimental.pallas import tpu_sc as plsc

BATCH = 16384
EMBED = 16
NUM = 1000000
_NC = 2
_NS = 16
_NW = _NC * _NS
_CHUNK = BATCH // _NW
_PASS = _CHUNK // 2

_SLAB = 1024
_SLABP = 1033
_NSLAB = 976
_MAIN = _NSLAB * _SLAB
_TAILW = 640
_TAIL0 = NUM - _TAILW


def _shuffle(slab_v, osl_v):
    row_iota = lax.iota(jnp.int32, 16)
    zeros = jnp.full((16,), 0, jnp.int32)

    def step(r, carry):
        c0 = r * 16
        vs = [plsc.load_gather(slab_v, [row_iota, zeros + (c0 + s)])
              for s in range(16)]
        for s in range(16):
            osl_v[r * 2 + (s >> 3), pl.ds((s & 7) * 16, 16)] = vs[s]
        return carry

    lax.fori_loop(0, _SLAB // 16, step, 0)


def _fire(uft_hbm, ift_hbm, u_b, i_b, sem, sl):
    c0 = sl * _SLAB
    pltpu.async_copy(
        uft_hbm.at[:, pl.ds(c0, _SLAB)], u_b.at[:, pl.ds(0, _SLAB)], sem)
    pltpu.async_copy(
        ift_hbm.at[:, pl.ds(c0, _SLAB)], i_b.at[:, pl.ds(0, _SLAB)], sem)


def _drain(uft_hbm, ift_hbm, u_b, i_b, sem, sl):
    c0 = sl * _SLAB
    pltpu.make_async_copy(
        uft_hbm.at[:, pl.ds(c0, _SLAB)], u_b.at[:, pl.ds(0, _SLAB)],
        sem).wait()
    pltpu.make_async_copy(
        ift_hbm.at[:, pl.ds(c0, _SLAB)], i_b.at[:, pl.ds(0, _SLAB)],
        sem).wait()


def _detrans_body(uft_hbm, ift_hbm, usup_hbm, isup_hbm,
                  u_s0, i_s0, u_s1, i_s1, uo_v, io_v, sem0, sem1, semw):
    wid = lax.axis_index("s") * _NC + lax.axis_index("c")
    bufs = ((u_s0, i_s0, sem0), (u_s1, i_s1, sem1))
    nrow = _SLAB // 8

    def _drain_writes(sl_prev):
        r0p = sl_prev * nrow
        pltpu.make_async_copy(
            uo_v, usup_hbm.at[pl.ds(r0p, nrow)], semw).wait()
        pltpu.make_async_copy(
            io_v, isup_hbm.at[pl.ds(r0p, nrow)], semw).wait()

    _fire(uft_hbm, ift_hbm, u_s0, i_s0, sem0, wid)

    def pair(k, carry):
        for e in range(2):
            j = k * 2 + e
            sl = j * _NW + wid
            u_b, i_b, sem = bufs[e]
            nu_b, ni_b, nsem = bufs[e ^ 1]

            @pl.when(sl < _NSLAB)
            def _(sl=sl, u_b=u_b, i_b=i_b, sem=sem,
                  nu_b=nu_b, ni_b=ni_b, nsem=nsem):
                _drain(uft_hbm, ift_hbm, u_b, i_b, sem, sl)

                @pl.when(sl + _NW < _NSLAB)
                def _():
                    _fire(uft_hbm, ift_hbm, nu_b, ni_b, nsem, sl + _NW)

                @pl.when(sl >= _NW)
                def _():
                    _drain_writes(sl - _NW)

                r0 = sl * nrow
                _shuffle(u_b, uo_v)
                pltpu.async_copy(uo_v, usup_hbm.at[pl.ds(r0, nrow)], semw)
                _shuffle(i_b, io_v)
                pltpu.async_copy(io_v, isup_hbm.at[pl.ds(r0, nrow)], semw)

        return carry

    lax.fori_loop(0, ((_NSLAB + _NW - 1) // _NW + 1) // 2, pair, 0)
    _drain_writes(((_NSLAB - 1 - wid) // _NW) * _NW + wid)


def _gather_body(uid_hbm, iid_hbm, uf_hbm, if_hbm, ut_hbm, it_hbm,
                 ub_hbm, ib_hbm, gb_hbm,
                 pred_hbm, cvr_hbm,
                 uid_v, iid_v, usup_v, isup_v, u_rows, i_rows, ut_v, it_v,
                 ub_v, ib_v, pred_v, cvr_v, gb_v, sem):
    wid = lax.axis_index("s") * _NC + lax.axis_index("c")
    base = wid * _CHUNK

    pltpu.sync_copy(uid_hbm.at[pl.ds(base, _CHUNK)], uid_v)
    pltpu.sync_copy(iid_hbm.at[pl.ds(base, _CHUNK)], iid_v)
    pltpu.sync_copy(gb_hbm, gb_v)
    pltpu.sync_copy(ut_hbm, ut_v)
    pltpu.sync_copy(it_hbm, it_v)

    def supidx(j, carry):
        s = j * 16
        usup_v[pl.ds(s, 16)] = lax.shift_right_logical(uid_v[pl.ds(s, 16)], 3)
        isup_v[pl.ds(s, 16)] = lax.shift_right_logical(iid_v[pl.ds(s, 16)], 3)
        return carry

    lax.fori_loop(0, _CHUNK // 16, supidx, 0)

    cp_ub = pltpu.async_copy(ub_hbm.at[uid_v], ub_v, sem)
    cp_ib = pltpu.async_copy(ib_hbm.at[iid_v], ib_v, sem)

    gb_vec = gb_v[...]

    for p in range(2):
        cp_u = pltpu.async_copy(
            uf_hbm.at[usup_v.at[pl.ds(p * _PASS, _PASS)]], u_rows, sem)
        cp_i = pltpu.async_copy(
            if_hbm.at[isup_v.at[pl.ds(p * _PASS, _PASS)]], i_rows, sem)
        cp_u.wait()
        cp_i.wait()
        if p == 0:
            cp_ub.wait()
            cp_ib.wait()

        def block(j, carry):
            b16 = p * _PASS + j * 16
            row_idx = lax.iota(jnp.int32, 16) + j * 16
            uid_b = uid_v[pl.ds(b16, 16)]
            iid_b = iid_v[pl.ds(b16, 16)]
            ucol = (uid_b & 7) * 16
            icol = (iid_b & 7) * 16
            ut_col = jnp.maximum(uid_b - _TAIL0, 0)
            it_col = jnp.maximum(iid_b - _TAIL0, 0)
            u_tail = uid_b >= _MAIN
            i_tail = iid_b >= _MAIN
            acc = ub_v[pl.ds(b16, 16)] + ib_v[pl.ds(b16, 16)] + gb_vec
            for d in range(EMBED):
                dvec = jnp.full((16,), d, jnp.int32)
                um = plsc.load_gather(u_rows, [row_idx, ucol + d])
                im = plsc.load_gather(i_rows, [row_idx, icol + d])
                ut = plsc.load_gather(ut_v, [dvec, ut_col])
                it = plsc.load_gather(it_v, [dvec, it_col])
                uu = jnp.where(u_tail, ut, um)
                ii = jnp.where(i_tail, it, im)
                acc = acc + uu * ii
            pred_v[pl.ds(b16, 16)] = acc
            cvr_v[pl.ds(b16, 16)] = 1.0 / (1.0 + jnp.exp(-acc))
            return carry

        lax.fori_loop(0, _PASS // 16, block, 0)

    pltpu.sync_copy(pred_v, pred_hbm.at[pl.ds(base, _CHUNK)])
    pltpu.sync_copy(cvr_v, cvr_hbm.at[pl.ds(base, _CHUNK)])


@jax.jit
def _run(user_id, item_id, user_factors, item_factors, user_bias, item_bias,
         gb16):
    f32 = jnp.float32
    mesh = plsc.VectorSubcoreMesh(core_axis_name="c", subcore_axis_name="s")
    params = pltpu.CompilerParams(
        needs_layout_passes=False, disable_bounds_checks=True)

    detrans = pl.kernel(
        _detrans_body,
        out_type=(jax.ShapeDtypeStruct((NUM // 8, 8 * EMBED), f32),
                  jax.ShapeDtypeStruct((NUM // 8, 8 * EMBED), f32)),
        mesh=mesh,
        compiler_params=params,
        scratch_types=[
            pltpu.VMEM((EMBED, _SLABP), f32),
            pltpu.VMEM((EMBED, _SLABP), f32),
            pltpu.VMEM((EMBED, _SLABP), f32),
            pltpu.VMEM((EMBED, _SLABP), f32),
            pltpu.VMEM((_SLAB // 8, 128), f32),
            pltpu.VMEM((_SLAB // 8, 128), f32),
            pltpu.SemaphoreType.DMA,
            pltpu.SemaphoreType.DMA,
            pltpu.SemaphoreType.DMA,
        ],
    )
    usup, isup = detrans(user_factors.T, item_factors.T)

    gather = pl.kernel(
        _gather_body,
        out_type=(jax.ShapeDtypeStruct((BATCH,), f32),
                  jax.ShapeDtypeStruct((BATCH,), f32)),
        mesh=mesh,
        compiler_params=params,
        scratch_types=[
            pltpu.VMEM((_CHUNK,), jnp.int32),
            pltpu.VMEM((_CHUNK,), jnp.int32),
            pltpu.VMEM((_CHUNK,), jnp.int32),
            pltpu.VMEM((_CHUNK,), jnp.int32),
            pltpu.VMEM((_PASS, 128), f32),
            pltpu.VMEM((_PASS, 128), f32),
            pltpu.VMEM((EMBED, _TAILW), f32),
            pltpu.VMEM((EMBED, _TAILW), f32),
            pltpu.VMEM((_CHUNK,), f32),
            pltpu.VMEM((_CHUNK,), f32),
            pltpu.VMEM((_CHUNK,), f32),
            pltpu.VMEM((_CHUNK,), f32),
            pltpu.VMEM((16,), f32),
            pltpu.SemaphoreType.DMA,
        ],
    )
    ut = user_factors[_TAIL0:].T
    it = item_factors[_TAIL0:].T
    return gather(user_id, item_id, usup, isup, ut, it,
                  user_bias, item_bias, gb16)


def kernel(user_id, item_id, user_factors, item_factors, user_bias,
           item_bias, global_bias):
    gb16 = jnp.broadcast_to(global_bias.astype(jnp.float32), (16,))
    pred, cvr = _run(user_id.astype(jnp.int32), item_id.astype(jnp.int32),
                     user_factors, item_factors, user_bias, item_bias, gb16)
    return (pred, cvr)
